# baseline (speedup 1.0000x reference)
import jax
import jax.numpy as jnp
from jax.experimental import pallas as pl
from jax.experimental.pallas import tpu as pltpu

D = 64


def _pick_block(n, candidates):
    for c in candidates:
        if n % c == 0:
            return c
    return n


def _feat_body(id_ref, fv_ref, ft_ref, wv_ref, wt_ref, bv_ref, bt_ref, out_ref):
    out_ref[:, 0:D] = id_ref[...]
    out_ref[:, D:2 * D] = (
        jnp.dot(fv_ref[...].astype(jnp.bfloat16), wv_ref[...].astype(jnp.bfloat16),
                preferred_element_type=jnp.float32)
        + bv_ref[...])
    out_ref[:, 2 * D:3 * D] = (
        jnp.dot(ft_ref[...].astype(jnp.bfloat16), wt_ref[...].astype(jnp.bfloat16),
                preferred_element_type=jnp.float32)
        + bt_ref[...])


def _compute_x(id_emb, feat_v, feat_t, wv, bv, wt, bt):
    n = id_emb.shape[0]
    vd = feat_v.shape[1]
    td = feat_t.shape[1]
    r = _pick_block(n, (400, 200, 80, 40, 16, 8))
    return pl.pallas_call(
        _feat_body,
        grid=(n // r,),
        in_specs=[
            pl.BlockSpec((r, D), lambda i: (i, 0)),
            pl.BlockSpec((r, vd), lambda i: (i, 0)),
            pl.BlockSpec((r, td), lambda i: (i, 0)),
            pl.BlockSpec((vd, D), lambda i: (0, 0)),
            pl.BlockSpec((td, D), lambda i: (0, 0)),
            pl.BlockSpec((1, D), lambda i: (0, 0)),
            pl.BlockSpec((1, D), lambda i: (0, 0)),
        ],
        out_specs=pl.BlockSpec((r, 3 * D), lambda i: (i, 0)),
        out_shape=jax.ShapeDtypeStruct((n, 3 * D), jnp.float32),
    )(id_emb, feat_v, feat_t, wv.T, wt.T, bv.reshape(1, D), bt.reshape(1, D))


def _attn1(x_q, x_kv, w):
    log2e = 1.4426950408889634
    qp, m = _attn_prep(x_q, x_kv, w)
    r = x_q.shape[0]
    rc = _pick_block(r, (16, 8))
    outs = []
    for c in range(0, r, rc):
        qc = qp[c:c + rc]
        mc = m[c:c + rc]
        xc = x_kv[c:c + rc]
        num = jnp.zeros((rc, D), jnp.float32)
        den = jnp.zeros((rc, D), jnp.float32)
        for t in range(D):
            xtb = xc[:, t:t + 1]
            e = jnp.exp2(qc * xtb - mc)
            num = num + e * xtb
            den = den + e
        outs.append(num / den)
    out = jnp.concatenate(outs, axis=0) if len(outs) > 1 else outs[0]
    return out * (w[2] * w[6]) + (w[5] * w[6] + w[7])


def _attn_prep(x_q, x_kv, w):
    log2e = 1.4426950408889634
    qp = (x_q * w[0] + w[3]) * (w[1] * log2e)
    xmax = jnp.max(x_kv, axis=-1, keepdims=True)
    xmin = jnp.min(x_kv, axis=-1, keepdims=True)
    m = jnp.maximum(qp * xmax, qp * xmin)
    return qp, m


def _attn_pair(xq_a, xkv_a, w_a, xq_b, xkv_b, w_b):
    qp_a, m_a = _attn_prep(xq_a, xkv_a, w_a)
    qp_b, m_b = _attn_prep(xq_b, xkv_b, w_b)
    r = xq_a.shape[0]
    rc = _pick_block(r, (40, 16, 8))
    outs_a = []
    outs_b = []
    for c in range(0, r, rc):
        q2 = jnp.concatenate([qp_a[c:c + rc], qp_b[c:c + rc]], axis=1)
        m2 = jnp.concatenate([m_a[c:c + rc], m_b[c:c + rc]], axis=1)
        xa = xkv_a[c:c + rc]
        xb = xkv_b[c:c + rc]
        num = jnp.zeros((rc, 2 * D), jnp.float32)
        den = jnp.zeros((rc, 2 * D), jnp.float32)
        for t in range(D):
            x2 = jnp.concatenate(
                [jnp.broadcast_to(xa[:, t:t + 1], (rc, D)),
                 jnp.broadcast_to(xb[:, t:t + 1], (rc, D))], axis=1)
            e = jnp.exp2(q2 * x2 - m2)
            num = num + e * x2
            den = den + e
        outs_a.append(num[:, :D] / den[:, :D])
        outs_b.append(num[:, D:] / den[:, D:])
    out_a = jnp.concatenate(outs_a, axis=0) if len(outs_a) > 1 else outs_a[0]
    out_b = jnp.concatenate(outs_b, axis=0) if len(outs_b) > 1 else outs_b[0]
    out_a = out_a * (w_a[2] * w_a[6]) + (w_a[5] * w_a[6] + w_a[7])
    out_b = out_b * (w_b[2] * w_b[6]) + (w_b[5] * w_b[6] + w_b[7])
    return out_a, out_b


def _ln_prelu(x, g, b, a):
    mu = jnp.mean(x, axis=-1, keepdims=True)
    xc = x - mu
    var = jnp.mean(xc * xc, axis=-1, keepdims=True)
    y = xc * jax.lax.rsqrt(var + 1e-5) * g + b
    return jnp.where(y >= 0.0, y, a * y)


def _l2norm_rows(y):
    nrm = jnp.sqrt(jnp.sum(y * y, axis=-1, keepdims=True))
    return y / jnp.maximum(nrm, 1e-12)


def _item_body(a_ref, x_ref, lng_ref, lnb_ref, sc_ref, id_ref, t2v_ref, v2t_ref):
    y = jnp.dot(a_ref[...].astype(jnp.bfloat16), x_ref[...],
                preferred_element_type=jnp.float32)
    y = _l2norm_rows(y)
    g = lng_ref[...]
    b = lnb_ref[...]
    alpha = sc_ref[4, 0]

    def w(row):
        return tuple(sc_ref[row, j] for j in range(8))

    gv = y[:, D:2 * D]
    gt = y[:, 2 * D:3 * D]
    id_ref[...] = y[:, 0:D]
    a1, a2 = _attn_pair(gv, gv, w(0), gt, gt, w(1))
    vfeat = _ln_prelu(gv + a1, g, b, alpha)
    tfeat = _ln_prelu(gt + a2, g, b, alpha)
    m1, m2 = _attn_pair(tfeat, vfeat, w(2), vfeat, tfeat, w(3))
    t2v_ref[...] = _ln_prelu(vfeat + m1, g, b, alpha)
    v2t_ref[...] = _ln_prelu(tfeat + m2, g, b, alpha)


def _user_body(a_ref, x_ref, sc_ref, id_ref, v_ref, t_ref):
    y = jnp.dot(a_ref[...].astype(jnp.bfloat16), x_ref[...],
                preferred_element_type=jnp.float32)
    y = _l2norm_rows(y)
    alpha = sc_ref[4, 0]
    id_ref[...] = y[:, 0:D]
    gv = y[:, D:2 * D]
    gt = y[:, 2 * D:3 * D]
    v_ref[...] = jnp.where(gv >= 0.0, gv, alpha * gv)
    t_ref[...] = jnp.where(gt >= 0.0, gt, alpha * gt)


def _prop_item(graph, x, lng, lnb, sc):
    n = graph.shape[0]
    k = graph.shape[1]
    r = _pick_block(n, (200, 80, 40, 16, 8))
    out = jax.ShapeDtypeStruct((n, D), jnp.float32)
    return pl.pallas_call(
        _item_body,
        grid=(n // r,),
        in_specs=[
            pl.BlockSpec((r, k), lambda i: (i, 0)),
            pl.BlockSpec((k, 3 * D), lambda i: (0, 0)),
            pl.BlockSpec((1, D), lambda i: (0, 0)),
            pl.BlockSpec((1, D), lambda i: (0, 0)),
            pl.BlockSpec(memory_space=pltpu.SMEM),
        ],
        out_specs=[pl.BlockSpec((r, D), lambda i: (i, 0))] * 3,
        out_shape=[out, out, out],
    )(graph, x, lng, lnb, sc)


def _prop_user(graph, x, sc):
    n = graph.shape[0]
    k = graph.shape[1]
    r = _pick_block(n, (400, 200, 80, 40, 16, 8))
    out = jax.ShapeDtypeStruct((n, D), jnp.float32)
    return pl.pallas_call(
        _user_body,
        grid=(n // r,),
        in_specs=[
            pl.BlockSpec((r, k), lambda i: (i, 0)),
            pl.BlockSpec((k, 3 * D), lambda i: (0, 0)),
            pl.BlockSpec(memory_space=pltpu.SMEM),
        ],
        out_specs=[pl.BlockSpec((r, D), lambda i: (i, 0))] * 3,
        out_shape=[out, out, out],
    )(graph, x, sc)


def _pack_scalars(p):
    rows = []
    for name in ('sa1', 'sa2', 'ma1', 'ma2'):
        q = p[name]
        rows.append(jnp.concatenate([q['in_w'], q['in_b'], q['out_w'], q['out_b']]))
    rows.append(jnp.full((8,), p['prelu_a'], jnp.float32))
    return jnp.stack(rows)


def kernel(params, ii_graph, uu_graph):
    p = params
    x_item = _compute_x(p['item_id_emb'], p['img_feat'], p['txt_feat'],
                        p['W_iv'], p['b_iv'], p['W_it'], p['b_it'])
    x_user = _compute_x(p['user_id_emb'], p['u_v_prefer'], p['u_t_prefer'],
                        p['W_uv'], p['b_uv'], p['W_ut'], p['b_ut'])
    sc = _pack_scalars(p)
    lng = p['ln_g'].reshape(1, D)
    lnb = p['ln_b'].reshape(1, D)
    item_id, t2v, v2t = _prop_item(ii_graph, x_item.astype(jnp.bfloat16), lng, lnb, sc)
    user_id, uv, ut = _prop_user(uu_graph, x_user.astype(jnp.bfloat16), sc)
    return (user_id, item_id, t2v, v2t, uv, ut)


# speedup vs baseline: 1.0807x; 1.0807x over previous
import jax
import jax.numpy as jnp
from jax.experimental import pallas as pl
from jax.experimental.pallas import tpu as pltpu

D = 64


def _pick_block(n, candidates):
    for c in candidates:
        if n % c == 0:
            return c
    return n


def _feat_body(id_ref, fv_ref, ft_ref, wv_ref, wt_ref, bv_ref, bt_ref, out_ref):
    out_ref[:, 0:D] = id_ref[...]
    out_ref[:, D:2 * D] = (
        jnp.dot(fv_ref[...].astype(jnp.bfloat16), wv_ref[...].astype(jnp.bfloat16),
                preferred_element_type=jnp.float32)
        + bv_ref[...])
    out_ref[:, 2 * D:3 * D] = (
        jnp.dot(ft_ref[...].astype(jnp.bfloat16), wt_ref[...].astype(jnp.bfloat16),
                preferred_element_type=jnp.float32)
        + bt_ref[...])


def _compute_x(id_emb, feat_v, feat_t, wv, bv, wt, bt):
    n = id_emb.shape[0]
    vd = feat_v.shape[1]
    td = feat_t.shape[1]
    r = _pick_block(n, (400, 200, 80, 40, 16, 8))
    return pl.pallas_call(
        _feat_body,
        grid=(n // r,),
        in_specs=[
            pl.BlockSpec((r, D), lambda i: (i, 0)),
            pl.BlockSpec((r, vd), lambda i: (i, 0)),
            pl.BlockSpec((r, td), lambda i: (i, 0)),
            pl.BlockSpec((vd, D), lambda i: (0, 0)),
            pl.BlockSpec((td, D), lambda i: (0, 0)),
            pl.BlockSpec((1, D), lambda i: (0, 0)),
            pl.BlockSpec((1, D), lambda i: (0, 0)),
        ],
        out_specs=pl.BlockSpec((r, 3 * D), lambda i: (i, 0)),
        out_shape=jax.ShapeDtypeStruct((n, 3 * D), jnp.float32),
    )(id_emb, feat_v, feat_t, wv.T, wt.T, bv.reshape(1, D), bt.reshape(1, D))


def _attn1(x_q, x_kv, w):
    log2e = 1.4426950408889634
    qp, m = _attn_prep(x_q, x_kv, w)
    r = x_q.shape[0]
    rc = _pick_block(r, (16, 8))
    outs = []
    for c in range(0, r, rc):
        qc = qp[c:c + rc]
        mc = m[c:c + rc]
        xc = x_kv[c:c + rc]
        num = jnp.zeros((rc, D), jnp.float32)
        den = jnp.zeros((rc, D), jnp.float32)
        for t in range(D):
            xtb = xc[:, t:t + 1]
            e = jnp.exp2(qc * xtb - mc)
            num = num + e * xtb
            den = den + e
        outs.append(num / den)
    out = jnp.concatenate(outs, axis=0) if len(outs) > 1 else outs[0]
    return out * (w[2] * w[6]) + (w[5] * w[6] + w[7])


def _attn_prep(x_q, x_kv, w):
    log2e = 1.4426950408889634
    qp = (x_q * w[0] + w[3]) * (w[1] * log2e)
    xmax = jnp.max(x_kv, axis=-1, keepdims=True)
    xmin = jnp.min(x_kv, axis=-1, keepdims=True)
    m = jnp.maximum(qp * xmax, qp * xmin)
    return qp, m


def _attn_pair(xq_a, xkv_a, w_a, xq_b, xkv_b, w_b):
    qp_a, m_a = _attn_prep(xq_a, xkv_a, w_a)
    qp_b, m_b = _attn_prep(xq_b, xkv_b, w_b)
    r = xq_a.shape[0]
    rc = _pick_block(r, (40, 16, 8))
    outs_a = []
    outs_b = []
    for c in range(0, r, rc):
        q2 = jnp.concatenate([qp_a[c:c + rc], qp_b[c:c + rc]], axis=1)
        m2 = jnp.concatenate([m_a[c:c + rc], m_b[c:c + rc]], axis=1)
        xa = xkv_a[c:c + rc]
        xb = xkv_b[c:c + rc]
        num = jnp.zeros((rc, 2 * D), jnp.float32)
        den = jnp.zeros((rc, 2 * D), jnp.float32)
        for t in range(D):
            x2 = jnp.concatenate(
                [jnp.broadcast_to(xa[:, t:t + 1], (rc, D)),
                 jnp.broadcast_to(xb[:, t:t + 1], (rc, D))], axis=1)
            e = jnp.exp2(q2 * x2 - m2)
            num = num + e * x2
            den = den + e
        outs_a.append(num[:, :D] / den[:, :D])
        outs_b.append(num[:, D:] / den[:, D:])
    out_a = jnp.concatenate(outs_a, axis=0) if len(outs_a) > 1 else outs_a[0]
    out_b = jnp.concatenate(outs_b, axis=0) if len(outs_b) > 1 else outs_b[0]
    out_a = out_a * (w_a[2] * w_a[6]) + (w_a[5] * w_a[6] + w_a[7])
    out_b = out_b * (w_b[2] * w_b[6]) + (w_b[5] * w_b[6] + w_b[7])
    return out_a, out_b


def _ln_prelu(x, g, b, a):
    mu = jnp.mean(x, axis=-1, keepdims=True)
    xc = x - mu
    var = jnp.mean(xc * xc, axis=-1, keepdims=True)
    y = xc * jax.lax.rsqrt(var + 1e-5) * g + b
    return jnp.where(y >= 0.0, y, a * y)


def _l2norm_rows(y):
    nrm = jnp.sqrt(jnp.sum(y * y, axis=-1, keepdims=True))
    return y / jnp.maximum(nrm, 1e-12)


def _attn_chain(gv, gt, lng_ref, lnb_ref, sc_ref):
    g = lng_ref[...]
    b = lnb_ref[...]
    alpha = sc_ref[4, 0]

    def w(row):
        return tuple(sc_ref[row, j] for j in range(8))

    a1, a2 = _attn_pair(gv, gv, w(0), gt, gt, w(1))
    vfeat = _ln_prelu(gv + a1, g, b, alpha)
    tfeat = _ln_prelu(gt + a2, g, b, alpha)
    m1, m2 = _attn_pair(tfeat, vfeat, w(2), vfeat, tfeat, w(3))
    t2v = _ln_prelu(vfeat + m1, g, b, alpha)
    v2t = _ln_prelu(tfeat + m2, g, b, alpha)
    return t2v, v2t


def _make_item_body(n_live_blocks):
    def _item_body(a_ref, x_ref, lng_ref, lnb_ref, sc_ref,
                   id_ref, t2v_ref, v2t_ref, raw_ref):
        y = jnp.dot(a_ref[...].astype(jnp.bfloat16), x_ref[...],
                    preferred_element_type=jnp.float32)
        y = _l2norm_rows(y)
        gv = y[:, D:2 * D]
        gt = y[:, 2 * D:3 * D]
        id_ref[...] = y[:, 0:D]
        raw_ref[:, 0:D] = gv
        raw_ref[:, D:2 * D] = gt

        @pl.when(pl.program_id(0) < n_live_blocks)
        def _():
            t2v, v2t = _attn_chain(gv, gt, lng_ref, lnb_ref, sc_ref)
            t2v_ref[...] = t2v
            v2t_ref[...] = v2t

    return _item_body


def _user_body(a_ref, x_ref, lng_ref, lnb_ref, sc_ref, raw_ref,
               id_ref, v_ref, t_ref, t2v_ref, v2t_ref):
    y = jnp.dot(a_ref[...].astype(jnp.bfloat16), x_ref[...],
                preferred_element_type=jnp.float32)
    y = _l2norm_rows(y)
    alpha = sc_ref[4, 0]
    id_ref[...] = y[:, 0:D]
    gv = y[:, D:2 * D]
    gt = y[:, 2 * D:3 * D]
    v_ref[...] = jnp.where(gv >= 0.0, gv, alpha * gv)
    t_ref[...] = jnp.where(gt >= 0.0, gt, alpha * gt)
    t2v, v2t = _attn_chain(raw_ref[:, 0:D], raw_ref[:, D:2 * D],
                           lng_ref, lnb_ref, sc_ref)
    t2v_ref[...] = t2v
    v2t_ref[...] = v2t


def _prop_item(graph, x, lng, lnb, sc, n_live_blocks):
    n = graph.shape[0]
    k = graph.shape[1]
    r = _pick_block(n, (200, 80, 40, 16, 8))
    out = jax.ShapeDtypeStruct((n, D), jnp.float32)
    raw = jax.ShapeDtypeStruct((n, 2 * D), jnp.float32)
    return pl.pallas_call(
        _make_item_body(n_live_blocks),
        grid=(n // r,),
        in_specs=[
            pl.BlockSpec((r, k), lambda i: (i, 0)),
            pl.BlockSpec((k, 3 * D), lambda i: (0, 0)),
            pl.BlockSpec((1, D), lambda i: (0, 0)),
            pl.BlockSpec((1, D), lambda i: (0, 0)),
            pl.BlockSpec(memory_space=pltpu.SMEM),
        ],
        out_specs=[pl.BlockSpec((r, D), lambda i: (i, 0))] * 3
        + [pl.BlockSpec((r, 2 * D), lambda i: (i, 0))],
        out_shape=[out, out, out, raw],
    )(graph, x, lng, lnb, sc)


def _prop_user(graph, x, lng, lnb, sc, raw, tail_start):
    n = graph.shape[0]
    k = graph.shape[1]
    n_items = raw.shape[0]
    r = _pick_block(n, (400, 200, 80, 40, 16, 8))
    grid = n // r
    tr = (n_items - tail_start) // grid
    tb = tail_start // tr
    out = jax.ShapeDtypeStruct((n, D), jnp.float32)
    out_i = jax.ShapeDtypeStruct((n_items, D), jnp.float32)
    return pl.pallas_call(
        _user_body,
        grid=(grid,),
        in_specs=[
            pl.BlockSpec((r, k), lambda i: (i, 0)),
            pl.BlockSpec((k, 3 * D), lambda i: (0, 0)),
            pl.BlockSpec((1, D), lambda i: (0, 0)),
            pl.BlockSpec((1, D), lambda i: (0, 0)),
            pl.BlockSpec(memory_space=pltpu.SMEM),
            pl.BlockSpec((tr, 2 * D), lambda i: (tb + i, 0)),
        ],
        out_specs=[pl.BlockSpec((r, D), lambda i: (i, 0))] * 3
        + [pl.BlockSpec((tr, D), lambda i: (tb + i, 0))] * 2,
        out_shape=[out, out, out, out_i, out_i],
    )(graph, x, lng, lnb, sc, raw)


def _pack_scalars(p):
    rows = []
    for name in ('sa1', 'sa2', 'ma1', 'ma2'):
        q = p[name]
        rows.append(jnp.concatenate([q['in_w'], q['in_b'], q['out_w'], q['out_b']]))
    rows.append(jnp.full((8,), p['prelu_a'], jnp.float32))
    return jnp.stack(rows)


def _user_plain_body(a_ref, x_ref, sc_ref, id_ref, v_ref, t_ref):
    y = jnp.dot(a_ref[...].astype(jnp.bfloat16), x_ref[...],
                preferred_element_type=jnp.float32)
    y = _l2norm_rows(y)
    alpha = sc_ref[4, 0]
    id_ref[...] = y[:, 0:D]
    gv = y[:, D:2 * D]
    gt = y[:, 2 * D:3 * D]
    v_ref[...] = jnp.where(gv >= 0.0, gv, alpha * gv)
    t_ref[...] = jnp.where(gt >= 0.0, gt, alpha * gt)


def _prop_user_plain(graph, x, sc):
    n = graph.shape[0]
    k = graph.shape[1]
    r = _pick_block(n, (400, 200, 80, 40, 16, 8))
    out = jax.ShapeDtypeStruct((n, D), jnp.float32)
    return pl.pallas_call(
        _user_plain_body,
        grid=(n // r,),
        in_specs=[
            pl.BlockSpec((r, k), lambda i: (i, 0)),
            pl.BlockSpec((k, 3 * D), lambda i: (0, 0)),
            pl.BlockSpec(memory_space=pltpu.SMEM),
        ],
        out_specs=[pl.BlockSpec((r, D), lambda i: (i, 0))] * 3,
        out_shape=[out, out, out],
    )(graph, x, sc)


def kernel(params, ii_graph, uu_graph):
    p = params
    x_item = _compute_x(p['item_id_emb'], p['img_feat'], p['txt_feat'],
                        p['W_iv'], p['b_iv'], p['W_it'], p['b_it'])
    x_user = _compute_x(p['user_id_emb'], p['u_v_prefer'], p['u_t_prefer'],
                        p['W_uv'], p['b_uv'], p['W_ut'], p['b_ut'])
    sc = _pack_scalars(p)
    lng = p['ln_g'].reshape(1, D)
    lnb = p['ln_b'].reshape(1, D)
    xi = x_item.astype(jnp.bfloat16)
    xu = x_user.astype(jnp.bfloat16)

    n_i = ii_graph.shape[0]
    r_i = _pick_block(n_i, (200, 80, 40, 16, 8))
    grid_i = n_i // r_i
    n_u = uu_graph.shape[0]
    r_u = _pick_block(n_u, (400, 200, 80, 40, 16, 8))
    grid_u = n_u // r_u
    defer = grid_i // 5
    tail = n_i - defer * r_i
    n_tail = n_i - tail
    ok = defer > 0 and n_tail % grid_u == 0
    if ok:
        tr = n_tail // grid_u
        ok = tr % 8 == 0 and tail % tr == 0
    if ok:
        item_id, t2v_a, v2t_a, raw = _prop_item(
            ii_graph, xi, lng, lnb, sc, grid_i - defer)
        user_id, uv, ut, t2v_b, v2t_b = _prop_user(
            uu_graph, xu, lng, lnb, sc, raw, tail)
        t2v = jnp.concatenate([t2v_a[:tail], t2v_b[tail:]], axis=0)
        v2t = jnp.concatenate([v2t_a[:tail], v2t_b[tail:]], axis=0)
    else:
        item_id, t2v, v2t, _ = _prop_item(ii_graph, xi, lng, lnb, sc, grid_i)
        user_id, uv, ut = _prop_user_plain(uu_graph, xu, sc)
    return (user_id, item_id, t2v, v2t, uv, ut)


# speedup vs baseline: 1.1168x; 1.0334x over previous
import jax
import jax.numpy as jnp
from jax.experimental import pallas as pl
from jax.experimental.pallas import tpu as pltpu

D = 64


def _pick_block(n, candidates):
    for c in candidates:
        if n % c == 0:
            return c
    return n


def _feat_body(id_ref, fv_ref, ft_ref, wv_ref, wt_ref, bv_ref, bt_ref, out_ref):
    out_ref[:, 0:D] = id_ref[...]
    out_ref[:, D:2 * D] = (
        jnp.dot(fv_ref[...].astype(jnp.bfloat16), wv_ref[...].astype(jnp.bfloat16),
                preferred_element_type=jnp.float32)
        + bv_ref[...])
    out_ref[:, 2 * D:3 * D] = (
        jnp.dot(ft_ref[...].astype(jnp.bfloat16), wt_ref[...].astype(jnp.bfloat16),
                preferred_element_type=jnp.float32)
        + bt_ref[...])


def _compute_x(id_emb, feat_v, feat_t, wv, bv, wt, bt):
    n = id_emb.shape[0]
    vd = feat_v.shape[1]
    td = feat_t.shape[1]
    r = _pick_block(n, (400, 200, 80, 40, 16, 8))
    return pl.pallas_call(
        _feat_body,
        grid=(n // r,),
        in_specs=[
            pl.BlockSpec((r, D), lambda i: (i, 0)),
            pl.BlockSpec((r, vd), lambda i: (i, 0)),
            pl.BlockSpec((r, td), lambda i: (i, 0)),
            pl.BlockSpec((vd, D), lambda i: (0, 0)),
            pl.BlockSpec((td, D), lambda i: (0, 0)),
            pl.BlockSpec((1, D), lambda i: (0, 0)),
            pl.BlockSpec((1, D), lambda i: (0, 0)),
        ],
        out_specs=pl.BlockSpec((r, 3 * D), lambda i: (i, 0)),
        out_shape=jax.ShapeDtypeStruct((n, 3 * D), jnp.float32),
    )(id_emb, feat_v, feat_t, wv.T, wt.T, bv.reshape(1, D), bt.reshape(1, D))


def _attn1(x_q, x_kv, w):
    log2e = 1.4426950408889634
    qp, m = _attn_prep(x_q, x_kv, w)
    r = x_q.shape[0]
    rc = _pick_block(r, (16, 8))
    outs = []
    for c in range(0, r, rc):
        qc = qp[c:c + rc]
        mc = m[c:c + rc]
        xc = x_kv[c:c + rc]
        num = jnp.zeros((rc, D), jnp.float32)
        den = jnp.zeros((rc, D), jnp.float32)
        for t in range(D):
            xtb = xc[:, t:t + 1]
            e = jnp.exp2(qc * xtb - mc)
            num = num + e * xtb
            den = den + e
        outs.append(num / den)
    out = jnp.concatenate(outs, axis=0) if len(outs) > 1 else outs[0]
    return out * (w[2] * w[6]) + (w[5] * w[6] + w[7])


def _attn_prep(x_q, x_kv, w):
    log2e = 1.4426950408889634
    qp = (x_q * w[0] + w[3]) * (w[1] * log2e)
    xmax = jnp.max(x_kv, axis=-1, keepdims=True)
    xmin = jnp.min(x_kv, axis=-1, keepdims=True)
    m = jnp.maximum(qp * xmax, qp * xmin)
    return qp, m


def _attn_pair(xq_a, xkv_a, w_a, xq_b, xkv_b, w_b):
    qp_a, m_a = _attn_prep(xq_a, xkv_a, w_a)
    qp_b, m_b = _attn_prep(xq_b, xkv_b, w_b)
    r = xq_a.shape[0]
    rc = _pick_block(r, (40, 16, 8))
    outs_a = []
    outs_b = []
    for c in range(0, r, rc):
        q2 = jnp.concatenate([qp_a[c:c + rc], qp_b[c:c + rc]], axis=1)
        m2 = jnp.concatenate([m_a[c:c + rc], m_b[c:c + rc]], axis=1)
        xa = xkv_a[c:c + rc]
        xb = xkv_b[c:c + rc]
        num = jnp.zeros((rc, 2 * D), jnp.float32)
        den = jnp.zeros((rc, 2 * D), jnp.float32)
        for t in range(D):
            x2 = jnp.concatenate(
                [jnp.broadcast_to(xa[:, t:t + 1], (rc, D)),
                 jnp.broadcast_to(xb[:, t:t + 1], (rc, D))], axis=1)
            e = jnp.exp2(q2 * x2 - m2)
            num = num + e * x2
            den = den + e
        outs_a.append(num[:, :D] / den[:, :D])
        outs_b.append(num[:, D:] / den[:, D:])
    out_a = jnp.concatenate(outs_a, axis=0) if len(outs_a) > 1 else outs_a[0]
    out_b = jnp.concatenate(outs_b, axis=0) if len(outs_b) > 1 else outs_b[0]
    out_a = out_a * (w_a[2] * w_a[6]) + (w_a[5] * w_a[6] + w_a[7])
    out_b = out_b * (w_b[2] * w_b[6]) + (w_b[5] * w_b[6] + w_b[7])
    return out_a, out_b


def _ln_prelu(x, g, b, a):
    mu = jnp.mean(x, axis=-1, keepdims=True)
    xc = x - mu
    var = jnp.mean(xc * xc, axis=-1, keepdims=True)
    y = xc * jax.lax.rsqrt(var + 1e-5) * g + b
    return jnp.where(y >= 0.0, y, a * y)


def _l2norm_rows(y):
    nrm = jnp.sqrt(jnp.sum(y * y, axis=-1, keepdims=True))
    return y / jnp.maximum(nrm, 1e-12)


def _attn_chain(gv, gt, lng_ref, lnb_ref, sc_ref):
    g = lng_ref[...]
    b = lnb_ref[...]
    alpha = sc_ref[4, 0]

    def w(row):
        return tuple(sc_ref[row, j] for j in range(8))

    a1, a2 = _attn_pair(gv, gv, w(0), gt, gt, w(1))
    vfeat = _ln_prelu(gv + a1, g, b, alpha)
    tfeat = _ln_prelu(gt + a2, g, b, alpha)
    m1, m2 = _attn_pair(tfeat, vfeat, w(2), vfeat, tfeat, w(3))
    t2v = _ln_prelu(vfeat + m1, g, b, alpha)
    v2t = _ln_prelu(tfeat + m2, g, b, alpha)
    return t2v, v2t


def _make_item_body(n_live_blocks):
    def _item_body(a_ref, x_ref, lng_ref, lnb_ref, sc_ref,
                   id_ref, t2v_ref, v2t_ref, raw_ref):
        y = jnp.dot(a_ref[...].astype(jnp.bfloat16), x_ref[...],
                    preferred_element_type=jnp.float32)
        y = _l2norm_rows(y)
        gv = y[:, D:2 * D]
        gt = y[:, 2 * D:3 * D]
        id_ref[...] = y[:, 0:D]
        raw_ref[:, 0:D] = gv
        raw_ref[:, D:2 * D] = gt

        @pl.when(pl.program_id(0) < n_live_blocks)
        def _():
            t2v, v2t = _attn_chain(gv, gt, lng_ref, lnb_ref, sc_ref)
            t2v_ref[...] = t2v
            v2t_ref[...] = v2t

    return _item_body


def _user_body(a_ref, x_ref, lng_ref, lnb_ref, sc_ref, raw_ref,
               id_ref, v_ref, t_ref, t2v_ref, v2t_ref):
    y = jnp.dot(a_ref[...].astype(jnp.bfloat16), x_ref[...],
                preferred_element_type=jnp.float32)
    y = _l2norm_rows(y)
    alpha = sc_ref[4, 0]
    id_ref[...] = y[:, 0:D]
    gv = y[:, D:2 * D]
    gt = y[:, 2 * D:3 * D]
    v_ref[...] = jnp.where(gv >= 0.0, gv, alpha * gv)
    t_ref[...] = jnp.where(gt >= 0.0, gt, alpha * gt)
    t2v, v2t = _attn_chain(raw_ref[:, 0:D], raw_ref[:, D:2 * D],
                           lng_ref, lnb_ref, sc_ref)
    t2v_ref[...] = t2v
    v2t_ref[...] = v2t


def _prop_item(graph, x, lng, lnb, sc, n_live_blocks):
    n = graph.shape[0]
    k = graph.shape[1]
    r = _pick_block(n, (400, 200, 80, 40, 16, 8))
    out = jax.ShapeDtypeStruct((n, D), jnp.float32)
    raw = jax.ShapeDtypeStruct((n, 2 * D), jnp.float32)
    return pl.pallas_call(
        _make_item_body(n_live_blocks),
        grid=(n // r,),
        in_specs=[
            pl.BlockSpec((r, k), lambda i: (i, 0)),
            pl.BlockSpec((k, 3 * D), lambda i: (0, 0)),
            pl.BlockSpec((1, D), lambda i: (0, 0)),
            pl.BlockSpec((1, D), lambda i: (0, 0)),
            pl.BlockSpec(memory_space=pltpu.SMEM),
        ],
        out_specs=[pl.BlockSpec((r, D), lambda i: (i, 0))] * 3
        + [pl.BlockSpec((r, 2 * D), lambda i: (i, 0))],
        out_shape=[out, out, out, raw],
    )(graph, x, lng, lnb, sc)


def _prop_user(graph, x, lng, lnb, sc, raw, tail_start):
    n = graph.shape[0]
    k = graph.shape[1]
    n_items = raw.shape[0]
    r = _pick_block(n, (400, 200, 80, 40, 16, 8))
    grid = n // r
    tr = (n_items - tail_start) // grid
    tb = tail_start // tr
    out = jax.ShapeDtypeStruct((n, D), jnp.float32)
    out_i = jax.ShapeDtypeStruct((n_items, D), jnp.float32)
    return pl.pallas_call(
        _user_body,
        grid=(grid,),
        in_specs=[
            pl.BlockSpec((r, k), lambda i: (i, 0)),
            pl.BlockSpec((k, 3 * D), lambda i: (0, 0)),
            pl.BlockSpec((1, D), lambda i: (0, 0)),
            pl.BlockSpec((1, D), lambda i: (0, 0)),
            pl.BlockSpec(memory_space=pltpu.SMEM),
            pl.BlockSpec((tr, 2 * D), lambda i: (tb + i, 0)),
        ],
        out_specs=[pl.BlockSpec((r, D), lambda i: (i, 0))] * 3
        + [pl.BlockSpec((tr, D), lambda i: (tb + i, 0))] * 2,
        out_shape=[out, out, out, out_i, out_i],
    )(graph, x, lng, lnb, sc, raw)


def _pack_scalars(p):
    rows = []
    for name in ('sa1', 'sa2', 'ma1', 'ma2'):
        q = p[name]
        rows.append(jnp.concatenate([q['in_w'], q['in_b'], q['out_w'], q['out_b']]))
    rows.append(jnp.full((8,), p['prelu_a'], jnp.float32))
    return jnp.stack(rows)


def _user_plain_body(a_ref, x_ref, sc_ref, id_ref, v_ref, t_ref):
    y = jnp.dot(a_ref[...].astype(jnp.bfloat16), x_ref[...],
                preferred_element_type=jnp.float32)
    y = _l2norm_rows(y)
    alpha = sc_ref[4, 0]
    id_ref[...] = y[:, 0:D]
    gv = y[:, D:2 * D]
    gt = y[:, 2 * D:3 * D]
    v_ref[...] = jnp.where(gv >= 0.0, gv, alpha * gv)
    t_ref[...] = jnp.where(gt >= 0.0, gt, alpha * gt)


def _prop_user_plain(graph, x, sc):
    n = graph.shape[0]
    k = graph.shape[1]
    r = _pick_block(n, (400, 200, 80, 40, 16, 8))
    out = jax.ShapeDtypeStruct((n, D), jnp.float32)
    return pl.pallas_call(
        _user_plain_body,
        grid=(n // r,),
        in_specs=[
            pl.BlockSpec((r, k), lambda i: (i, 0)),
            pl.BlockSpec((k, 3 * D), lambda i: (0, 0)),
            pl.BlockSpec(memory_space=pltpu.SMEM),
        ],
        out_specs=[pl.BlockSpec((r, D), lambda i: (i, 0))] * 3,
        out_shape=[out, out, out],
    )(graph, x, sc)


def kernel(params, ii_graph, uu_graph):
    p = params
    x_item = _compute_x(p['item_id_emb'], p['img_feat'], p['txt_feat'],
                        p['W_iv'], p['b_iv'], p['W_it'], p['b_it'])
    x_user = _compute_x(p['user_id_emb'], p['u_v_prefer'], p['u_t_prefer'],
                        p['W_uv'], p['b_uv'], p['W_ut'], p['b_ut'])
    sc = _pack_scalars(p)
    lng = p['ln_g'].reshape(1, D)
    lnb = p['ln_b'].reshape(1, D)
    xi = x_item.astype(jnp.bfloat16)
    xu = x_user.astype(jnp.bfloat16)

    n_i = ii_graph.shape[0]
    r_i = _pick_block(n_i, (400, 200, 80, 40, 16, 8))
    grid_i = n_i // r_i
    n_u = uu_graph.shape[0]
    r_u = _pick_block(n_u, (400, 200, 80, 40, 16, 8))
    grid_u = n_u // r_u
    defer = grid_i // 5
    tail = n_i - defer * r_i
    n_tail = n_i - tail
    ok = defer > 0 and n_tail % grid_u == 0
    if ok:
        tr = n_tail // grid_u
        ok = tr % 8 == 0 and tail % tr == 0
    if ok:
        item_id, t2v_a, v2t_a, raw = _prop_item(
            ii_graph, xi, lng, lnb, sc, grid_i - defer)
        user_id, uv, ut, t2v_b, v2t_b = _prop_user(
            uu_graph, xu, lng, lnb, sc, raw, tail)
        t2v = jnp.concatenate([t2v_a[:tail], t2v_b[tail:]], axis=0)
        v2t = jnp.concatenate([v2t_a[:tail], v2t_b[tail:]], axis=0)
    else:
        item_id, t2v, v2t, _ = _prop_item(ii_graph, xi, lng, lnb, sc, grid_i)
        user_id, uv, ut = _prop_user_plain(uu_graph, xu, sc)
    return (user_id, item_id, t2v, v2t, uv, ut)


# speedup vs baseline: 1.1179x; 1.0009x over previous
import jax
import jax.numpy as jnp
from jax.experimental import pallas as pl
from jax.experimental.pallas import tpu as pltpu

D = 64


def _pick_block(n, candidates):
    for c in candidates:
        if n % c == 0:
            return c
    return n


def _feat_body(id_ref, fv_ref, ft_ref, wv_ref, wt_ref, bv_ref, bt_ref, out_ref):
    out_ref[:, 0:D] = id_ref[...]
    out_ref[:, D:2 * D] = (
        jnp.dot(fv_ref[...].astype(jnp.bfloat16), wv_ref[...].astype(jnp.bfloat16),
                preferred_element_type=jnp.float32)
        + bv_ref[...])
    out_ref[:, 2 * D:3 * D] = (
        jnp.dot(ft_ref[...].astype(jnp.bfloat16), wt_ref[...].astype(jnp.bfloat16),
                preferred_element_type=jnp.float32)
        + bt_ref[...])


def _compute_x(id_emb, feat_v, feat_t, wv, bv, wt, bt):
    n = id_emb.shape[0]
    vd = feat_v.shape[1]
    td = feat_t.shape[1]
    r = _pick_block(n, (400, 200, 80, 40, 16, 8))
    return pl.pallas_call(
        _feat_body,
        grid=(n // r,),
        in_specs=[
            pl.BlockSpec((r, D), lambda i: (i, 0)),
            pl.BlockSpec((r, vd), lambda i: (i, 0)),
            pl.BlockSpec((r, td), lambda i: (i, 0)),
            pl.BlockSpec((vd, D), lambda i: (0, 0)),
            pl.BlockSpec((td, D), lambda i: (0, 0)),
            pl.BlockSpec((1, D), lambda i: (0, 0)),
            pl.BlockSpec((1, D), lambda i: (0, 0)),
        ],
        out_specs=pl.BlockSpec((r, 3 * D), lambda i: (i, 0)),
        out_shape=jax.ShapeDtypeStruct((n, 3 * D), jnp.float32),
    )(id_emb, feat_v, feat_t, wv.T, wt.T, bv.reshape(1, D), bt.reshape(1, D))


def _attn_prep(x_q, x_kv, w):
    log2e = 1.4426950408889634
    qp = (x_q * w[0] + w[3]) * (w[1] * log2e)
    xmax = jnp.max(x_kv, axis=-1, keepdims=True)
    xmin = jnp.min(x_kv, axis=-1, keepdims=True)
    m = jnp.maximum(qp * xmax, qp * xmin)
    return qp, m


def _attn_pair(xq_a, xkv_a, w_a, xq_b, xkv_b, w_b):
    qp_a, m_a = _attn_prep(xq_a, xkv_a, w_a)
    qp_b, m_b = _attn_prep(xq_b, xkv_b, w_b)
    r = xq_a.shape[0]
    rc = _pick_block(r, (40, 16, 8))
    outs_a = []
    outs_b = []
    for c in range(0, r, rc):
        q2 = jnp.concatenate([qp_a[c:c + rc], qp_b[c:c + rc]], axis=1)
        m2 = jnp.concatenate([m_a[c:c + rc], m_b[c:c + rc]], axis=1)
        xa = xkv_a[c:c + rc]
        xb = xkv_b[c:c + rc]
        num = jnp.zeros((rc, 2 * D), jnp.float32)
        den = jnp.zeros((rc, 2 * D), jnp.float32)
        for t in range(D):
            x2 = jnp.concatenate(
                [jnp.broadcast_to(xa[:, t:t + 1], (rc, D)),
                 jnp.broadcast_to(xb[:, t:t + 1], (rc, D))], axis=1)
            e = jnp.exp2(q2 * x2 - m2)
            num = num + e * x2
            den = den + e
        outs_a.append(num[:, :D] / den[:, :D])
        outs_b.append(num[:, D:] / den[:, D:])
    out_a = jnp.concatenate(outs_a, axis=0) if len(outs_a) > 1 else outs_a[0]
    out_b = jnp.concatenate(outs_b, axis=0) if len(outs_b) > 1 else outs_b[0]
    out_a = out_a * (w_a[2] * w_a[6]) + (w_a[5] * w_a[6] + w_a[7])
    out_b = out_b * (w_b[2] * w_b[6]) + (w_b[5] * w_b[6] + w_b[7])
    return out_a, out_b


def _ln_prelu(x, g, b, a):
    mu = jnp.mean(x, axis=-1, keepdims=True)
    xc = x - mu
    var = jnp.mean(xc * xc, axis=-1, keepdims=True)
    y = xc * jax.lax.rsqrt(var + 1e-5) * g + b
    return jnp.where(y >= 0.0, y, a * y)


def _l2norm_rows(y):
    nrm = jnp.sqrt(jnp.sum(y * y, axis=-1, keepdims=True))
    return y / jnp.maximum(nrm, 1e-12)


def _attn_chain(gv, gt, lng_ref, lnb_ref, sc_ref):
    g = lng_ref[...]
    b = lnb_ref[...]
    alpha = sc_ref[4, 0]

    def w(row):
        return tuple(sc_ref[row, j] for j in range(8))

    a1, a2 = _attn_pair(gv, gv, w(0), gt, gt, w(1))
    vfeat = _ln_prelu(gv + a1, g, b, alpha)
    tfeat = _ln_prelu(gt + a2, g, b, alpha)
    m1, m2 = _attn_pair(tfeat, vfeat, w(2), vfeat, tfeat, w(3))
    t2v = _ln_prelu(vfeat + m1, g, b, alpha)
    v2t = _ln_prelu(tfeat + m2, g, b, alpha)
    return t2v, v2t


def _make_item_body(n_live_blocks):
    def _item_body(a_ref, x_ref, lng_ref, lnb_ref, sc_ref,
                   id_ref, t2v_ref, v2t_ref, raw_ref):
        y = jnp.dot(a_ref[...].astype(jnp.bfloat16), x_ref[...],
                    preferred_element_type=jnp.float32)
        y = _l2norm_rows(y)
        gv = y[:, D:2 * D]
        gt = y[:, 2 * D:3 * D]
        id_ref[...] = y[:, 0:D]
        raw_ref[:, 0:D] = gv
        raw_ref[:, D:2 * D] = gt

        @pl.when(pl.program_id(0) < n_live_blocks)
        def _():
            t2v, v2t = _attn_chain(gv, gt, lng_ref, lnb_ref, sc_ref)
            t2v_ref[...] = t2v
            v2t_ref[...] = v2t

    return _item_body


def _user_body(a_ref, x_ref, lng_ref, lnb_ref, sc_ref, raw_ref,
               id_ref, v_ref, t_ref, t2v_ref, v2t_ref):
    y = jnp.dot(a_ref[...].astype(jnp.bfloat16), x_ref[...],
                preferred_element_type=jnp.float32)
    y = _l2norm_rows(y)
    alpha = sc_ref[4, 0]
    id_ref[...] = y[:, 0:D]
    gv = y[:, D:2 * D]
    gt = y[:, 2 * D:3 * D]
    v_ref[...] = jnp.where(gv >= 0.0, gv, alpha * gv)
    t_ref[...] = jnp.where(gt >= 0.0, gt, alpha * gt)
    t2v, v2t = _attn_chain(raw_ref[:, 0:D], raw_ref[:, D:2 * D],
                           lng_ref, lnb_ref, sc_ref)
    t2v_ref[...] = t2v
    v2t_ref[...] = v2t


def _prop_item(graph, x, lng, lnb, sc, n_live_blocks):
    n = graph.shape[0]
    k = graph.shape[1]
    r = _pick_block(n, (400, 200, 80, 40, 16, 8))
    out = jax.ShapeDtypeStruct((n, D), jnp.float32)
    raw = jax.ShapeDtypeStruct((n, 2 * D), jnp.float32)
    return pl.pallas_call(
        _make_item_body(n_live_blocks),
        grid=(n // r,),
        in_specs=[
            pl.BlockSpec((r, k), lambda i: (i, 0)),
            pl.BlockSpec((k, 3 * D), lambda i: (0, 0)),
            pl.BlockSpec((1, D), lambda i: (0, 0)),
            pl.BlockSpec((1, D), lambda i: (0, 0)),
            pl.BlockSpec(memory_space=pltpu.SMEM),
        ],
        out_specs=[pl.BlockSpec((r, D), lambda i: (i, 0))] * 3
        + [pl.BlockSpec((r, 2 * D), lambda i: (i, 0))],
        out_shape=[out, out, out, raw],
    )(graph, x, lng, lnb, sc)


def _prop_user(graph, x, lng, lnb, sc, raw, tail_start):
    n = graph.shape[0]
    k = graph.shape[1]
    n_items = raw.shape[0]
    r = _pick_block(n, (400, 200, 80, 40, 16, 8))
    grid = n // r
    tr = (n_items - tail_start) // grid
    tb = tail_start // tr
    out = jax.ShapeDtypeStruct((n, D), jnp.float32)
    out_i = jax.ShapeDtypeStruct((n_items, D), jnp.float32)
    return pl.pallas_call(
        _user_body,
        grid=(grid,),
        in_specs=[
            pl.BlockSpec((r, k), lambda i: (i, 0)),
            pl.BlockSpec((k, 3 * D), lambda i: (0, 0)),
            pl.BlockSpec((1, D), lambda i: (0, 0)),
            pl.BlockSpec((1, D), lambda i: (0, 0)),
            pl.BlockSpec(memory_space=pltpu.SMEM),
            pl.BlockSpec((tr, 2 * D), lambda i: (tb + i, 0)),
        ],
        out_specs=[pl.BlockSpec((r, D), lambda i: (i, 0))] * 3
        + [pl.BlockSpec((tr, D), lambda i: (tb + i, 0))] * 2,
        out_shape=[out, out, out, out_i, out_i],
    )(graph, x, lng, lnb, sc, raw)


def _pack_scalars(p):
    rows = []
    for name in ('sa1', 'sa2', 'ma1', 'ma2'):
        q = p[name]
        rows.append(jnp.concatenate([q['in_w'], q['in_b'], q['out_w'], q['out_b']]))
    rows.append(jnp.full((8,), p['prelu_a'], jnp.float32))
    return jnp.stack(rows)


def _user_plain_body(a_ref, x_ref, sc_ref, id_ref, v_ref, t_ref):
    y = jnp.dot(a_ref[...].astype(jnp.bfloat16), x_ref[...],
                preferred_element_type=jnp.float32)
    y = _l2norm_rows(y)
    alpha = sc_ref[4, 0]
    id_ref[...] = y[:, 0:D]
    gv = y[:, D:2 * D]
    gt = y[:, 2 * D:3 * D]
    v_ref[...] = jnp.where(gv >= 0.0, gv, alpha * gv)
    t_ref[...] = jnp.where(gt >= 0.0, gt, alpha * gt)


def _prop_user_plain(graph, x, sc):
    n = graph.shape[0]
    k = graph.shape[1]
    r = _pick_block(n, (400, 200, 80, 40, 16, 8))
    out = jax.ShapeDtypeStruct((n, D), jnp.float32)
    return pl.pallas_call(
        _user_plain_body,
        grid=(n // r,),
        in_specs=[
            pl.BlockSpec((r, k), lambda i: (i, 0)),
            pl.BlockSpec((k, 3 * D), lambda i: (0, 0)),
            pl.BlockSpec(memory_space=pltpu.SMEM),
        ],
        out_specs=[pl.BlockSpec((r, D), lambda i: (i, 0))] * 3,
        out_shape=[out, out, out],
    )(graph, x, sc)


def kernel(params, ii_graph, uu_graph):
    p = params
    x_item = _compute_x(p['item_id_emb'], p['img_feat'], p['txt_feat'],
                        p['W_iv'], p['b_iv'], p['W_it'], p['b_it'])
    x_user = _compute_x(p['user_id_emb'], p['u_v_prefer'], p['u_t_prefer'],
                        p['W_uv'], p['b_uv'], p['W_ut'], p['b_ut'])
    sc = _pack_scalars(p)
    lng = p['ln_g'].reshape(1, D)
    lnb = p['ln_b'].reshape(1, D)
    xi = x_item.astype(jnp.bfloat16)
    xu = x_user.astype(jnp.bfloat16)

    n_i = ii_graph.shape[0]
    r_i = _pick_block(n_i, (400, 200, 80, 40, 16, 8))
    grid_i = n_i // r_i
    n_u = uu_graph.shape[0]
    r_u = _pick_block(n_u, (400, 200, 80, 40, 16, 8))
    grid_u = n_u // r_u
    defer = grid_i // 5
    tail = n_i - defer * r_i
    n_tail = n_i - tail
    ok = defer > 0 and n_tail % grid_u == 0
    if ok:
        tr = n_tail // grid_u
        ok = tr % 8 == 0 and tail % tr == 0
    if ok:
        item_id, t2v_a, v2t_a, raw = _prop_item(
            ii_graph, xi, lng, lnb, sc, grid_i - defer)
        user_id, uv, ut, t2v_b, v2t_b = _prop_user(
            uu_graph, xu, lng, lnb, sc, raw, tail)
        t2v = jnp.concatenate([t2v_a[:tail], t2v_b[tail:]], axis=0)
        v2t = jnp.concatenate([v2t_a[:tail], v2t_b[tail:]], axis=0)
    else:
        item_id, t2v, v2t, _ = _prop_item(ii_graph, xi, lng, lnb, sc, grid_i)
        user_id, uv, ut = _prop_user_plain(uu_graph, xu, sc)
    return (user_id, item_id, t2v, v2t, uv, ut)
